# baseline (speedup 1.0000x reference)
import functools

import jax
import jax.numpy as jnp
from jax.experimental import pallas as pl
from jax.experimental.pallas import tpu as pltpu

B = 16
D = 32
HID = 128
POS = 512 * 512
CHP = 8192
NSTEPS = POS // CHP
TOTAL = B * NSTEPS
NBUF = 8
ACC = 256


def _reduce_body(x_hbm, acc_ref, buf, sem):
    b = pl.program_id(0)
    j = pl.program_id(1)
    k = b * NSTEPS + j
    view = x_hbm.reshape(B, POS, D)

    def start(step):
        bb = step // NSTEPS
        jj = step % NSTEPS
        slot = step % NBUF
        pltpu.make_async_copy(
            view.at[bb, pl.ds(jj * CHP, CHP), :], buf.at[slot], sem.at[slot]
        ).start()

    @pl.when(k == 0)
    def _():
        for i in range(NBUF - 1):
            start(i)

    @pl.when(k + NBUF - 1 < TOTAL)
    def _():
        start(k + NBUF - 1)

    pltpu.make_async_copy(
        view.at[b, pl.ds(j * CHP, CHP), :], buf.at[k % NBUF], sem.at[k % NBUF]
    ).wait()

    @pl.when(j == 0)
    def _():
        acc_ref[...] = jnp.zeros_like(acc_ref)

    x = buf[k % NBUF]
    acc_ref[0] += jnp.sum(x.reshape(CHP // ACC, ACC, D), axis=0)


def _head_body(p_ref, w1_ref, b1_ref, w2_ref, b2_ref, g_ref,
               rout_ref, probs_ref, idx_ref):
    p = jnp.sum(p_ref[...], axis=1)
    x = p * (1.0 / POS)
    h = jnp.maximum(
        jax.lax.dot_general(x, w1_ref[...], (((1,), (0,)), ((), ())),
                            preferred_element_type=jnp.float32) + b1_ref[...],
        0.0)
    logits = jax.lax.dot_general(h, w2_ref[...], (((1,), (0,)), ((), ())),
                                 preferred_element_type=jnp.float32) + b2_ref[...]
    m = jnp.max(logits, axis=-1, keepdims=True)
    e = jnp.exp(logits - m)
    probs = e / jnp.sum(e, axis=-1, keepdims=True)
    probs_ref[...] = probs
    z = jnp.log(probs + 1e-20) + g_ref[...]
    zmax = jnp.max(z, axis=-1, keepdims=True)
    lane = jax.lax.broadcasted_iota(jnp.int32, (B, D), 1)
    idx = jnp.min(jnp.where(z >= zmax, lane, D), axis=-1, keepdims=True)
    idx_ref[...] = idx
    rout_ref[...] = (lane == idx).astype(jnp.float32)


@functools.partial(jax.jit, static_argnames=())
def kernel(attn_5d, W1, b1, W2, b2):
    partial = pl.pallas_call(
        _reduce_body,
        grid=(B, NSTEPS),
        in_specs=[pl.BlockSpec(memory_space=pltpu.MemorySpace.HBM)],
        out_specs=pl.BlockSpec((1, ACC, D), lambda b, j: (b, 0, 0)),
        out_shape=jax.ShapeDtypeStruct((B, ACC, D), jnp.float32),
        scratch_shapes=[
            pltpu.VMEM((NBUF, CHP, D), jnp.float32),
            pltpu.SemaphoreType.DMA((NBUF,)),
        ],
    )(attn_5d)

    gumbel = jax.random.gumbel(jax.random.key(1234), (B, D), jnp.float32)
    rout, probs, idx = pl.pallas_call(
        _head_body,
        in_specs=[
            pl.BlockSpec((B, ACC, D), lambda: (0, 0, 0)),
            pl.BlockSpec((D, HID), lambda: (0, 0)),
            pl.BlockSpec((1, HID), lambda: (0, 0)),
            pl.BlockSpec((HID, D), lambda: (0, 0)),
            pl.BlockSpec((1, D), lambda: (0, 0)),
            pl.BlockSpec((B, D), lambda: (0, 0)),
        ],
        out_specs=[
            pl.BlockSpec((B, D), lambda: (0, 0)),
            pl.BlockSpec((B, D), lambda: (0, 0)),
            pl.BlockSpec((B, 1), lambda: (0, 0)),
        ],
        out_shape=[
            jax.ShapeDtypeStruct((B, D), jnp.float32),
            jax.ShapeDtypeStruct((B, D), jnp.float32),
            jax.ShapeDtypeStruct((B, 1), jnp.int32),
        ],
    )(partial, W1, b1.reshape(1, HID), W2, b2.reshape(1, D), gumbel)
    return rout, probs, idx.reshape(B)
